# baseline (speedup 1.0000x reference)
import functools

import jax
import jax.numpy as jnp
from jax.experimental import pallas as pl


def _mm_step(i, blk, bit, xa_ref, xb_ref, w1_ref, b1_ref, w2_ref, b2_ref,
             key_ref, k31_ref, tx_ref, out_ref):
    fa = jnp.maximum(
        jax.lax.dot_general(
            xa_ref[...], w1_ref[...], (((1,), (0,)), ((), ())),
            preferred_element_type=jnp.float32,
        ) + b1_ref[...], 0.0)
    fb = jnp.maximum(
        jax.lax.dot_general(
            xb_ref[...], w1_ref[...], (((1,), (0,)), ((), ())),
            preferred_element_type=jnp.float32,
        ) + b1_ref[...], 0.0)
    ha = jax.lax.dot_general(
        fa, w2_ref[...], (((1,), (0,)), ((), ())),
        preferred_element_type=jnp.float32,
    ) + b2_ref[...]
    hb = jax.lax.dot_general(
        fb, w2_ref[...], (((1,), (0,)), ((), ())),
        preferred_element_type=jnp.float32,
    ) + b2_ref[...]
    ka, kb = [], []
    for h in (ha, hb):
        i32 = jax.lax.bitcast_convert_type(h, jnp.int32)
        s32 = jnp.right_shift(i32, 31)
        k = (i32 ^ (s32 & jnp.int32(0x7FFFFFFF))) - s32
        ka.append(k)
        kb.append(jax.lax.shift_right_logical(k ^ jnp.int32(-(2**31)), 1))
    key_ref[pl.ds(i * blk, blk), :bit] = ka[0]
    key_ref[pl.ds(i * blk, blk), bit:] = ka[1]
    k31_ref[pl.ds(i * blk, blk), :bit] = kb[0]
    k31_ref[pl.ds(i * blk, blk), bit:] = kb[1]
    num = jnp.sum(fa * fb, axis=1, keepdims=True)
    na = jnp.maximum(jnp.sqrt(jnp.sum(fa * fa, axis=1, keepdims=True)), 1e-8)
    nb = jnp.maximum(jnp.sqrt(jnp.sum(fb * fb, axis=1, keepdims=True)), 1e-8)
    tx_ref[pl.ds(i * blk, blk), :] = num / (na * nb)


def _hash_loss_step(n2, bit, key_ref, k31_ref, tx_ref, out_ref):
    n = 2 * n2
    int_min = jnp.int32(-(2**31))
    key = key_ref[...]
    k31 = k31_ref[...]

    def fold(s):
        return s[:, :bit] + s[:, bit:]

    def tree(s):
        r = s.shape[0]
        while r > 8:
            r //= 2
            s = s[:r] + s[r:]
        return jnp.sum(s, axis=0, keepdims=True)

    def neg_count_lt(arr, p2):
        return fold(tree(jax.lax.shift_right_arithmetic(arr - p2, 31)))

    def both(v):
        return jnp.concatenate([v, v], axis=1)

    p = jnp.zeros((1, bit), jnp.int32)
    for bb in range(30, -1, -1):
        pp = p | jnp.int32(1 << bb)
        s = neg_count_lt(k31, both(pp))
        p = jnp.where(s >= -n2, pp, p)

    cnt_hi = n + neg_count_lt(k31, both(p + 1))
    eqm = jax.lax.shift_right_arithmetic((k31 ^ both(p)) - 1, 31)
    lsbm = -(key & 1)
    cnt_eq1 = -fold(tree(eqm & lsbm))
    lsb = jnp.where(cnt_hi + cnt_eq1 >= n2, jnp.int32(1), jnp.int32(0))
    thr = both((jnp.left_shift(p, 1) | lsb) ^ int_min)

    greater = key > thr
    g = jnp.where(lsb == 1, cnt_hi, cnt_hi + cnt_eq1)
    m = n2 - g
    eq = key == thr

    prow = jax.lax.broadcasted_iota(jnp.int32, (n2, 2 * bit), 0)
    lane = jax.lax.broadcasted_iota(jnp.int32, (n2, 2 * bit), 1)
    row = prow + jnp.where(lane >= bit, n2, 0)

    eqrow = jnp.where(eq, row, jnp.int32(1 << 30))

    def tie_min(s):
        r = s.shape[0]
        while r > 8:
            r //= 2
            s = jnp.minimum(s[:r], s[r:])
        s = jnp.min(s, axis=0, keepdims=True)
        return jnp.minimum(s[:, :bit], s[:, bit:])

    def tie_search(s):
        q = jnp.zeros((1, bit), jnp.int32)
        for bb in range(11, -1, -1):
            qp = q | jnp.int32(1 << bb)
            c = neg_count_lt(eqrow, both(qp))
            q = jnp.where(c > -m, qp, q)
        return q

    q = jax.lax.cond(jnp.all(m == 1), tie_min, tie_search, eqrow)

    plus = greater | (eq & (row <= both(q)))
    ba = jnp.where(plus[:, :bit], 1.0, -1.0)
    bb_ = jnp.where(plus[:, bit:], 1.0, -1.0)

    tb = jnp.sum(ba * bb_, axis=1, keepdims=True) / float(bit)

    diff = tb - tx_ref[...]
    out_ref[...] = jnp.sum(diff * diff, axis=0, keepdims=True) / float(n2)


def _fused_kernel(nblk, blk, n2, bit, xa_ref, xb_ref, w1_ref, b1_ref, w2_ref,
                  b2_ref, out_ref, key_ref, k31_ref, tx_ref):
    i = pl.program_id(0)

    @pl.when(i < nblk)
    def _():
        _mm_step(i, blk, bit, xa_ref, xb_ref, w1_ref, b1_ref, w2_ref, b2_ref,
                 key_ref, k31_ref, tx_ref, out_ref)

    @pl.when(i == nblk)
    def _():
        _hash_loss_step(n2, bit, key_ref, k31_ref, tx_ref, out_ref)


def kernel(x, W1, b1, W2, b2):
    n, d = x.shape
    hid = W1.shape[1]
    bit = W2.shape[1]
    n2 = n // 2
    blk = 512
    nblk = n2 // blk
    last = nblk - 1

    loss, _, _, _ = pl.pallas_call(
        functools.partial(_fused_kernel, nblk, blk, n2, bit),
        grid=(nblk + 1,),
        in_specs=[
            pl.BlockSpec((blk, d), lambda i: (jnp.minimum(i, last), 0)),
            pl.BlockSpec((blk, d), lambda i: (jnp.minimum(i, last) + nblk, 0)),
            pl.BlockSpec((d, hid), lambda i: (0, 0)),
            pl.BlockSpec((1, hid), lambda i: (0, 0)),
            pl.BlockSpec((hid, bit), lambda i: (0, 0)),
            pl.BlockSpec((1, bit), lambda i: (0, 0)),
        ],
        out_specs=[
            pl.BlockSpec((1, 1), lambda i: (0, 0)),
            pl.BlockSpec((n2, 2 * bit), lambda i: (0, 0)),
            pl.BlockSpec((n2, 2 * bit), lambda i: (0, 0)),
            pl.BlockSpec((n2, 1), lambda i: (0, 0)),
        ],
        out_shape=[
            jax.ShapeDtypeStruct((1, 1), jnp.float32),
            jax.ShapeDtypeStruct((n2, 2 * bit), jnp.int32),
            jax.ShapeDtypeStruct((n2, 2 * bit), jnp.int32),
            jax.ShapeDtypeStruct((n2, 1), jnp.float32),
        ],
    )(x, x, W1, b1.reshape(1, hid), W2, b2.reshape(1, bit))
    return loss[0, 0]


# speedup vs baseline: 1.0058x; 1.0058x over previous
import functools

import jax
import jax.numpy as jnp
from jax.experimental import pallas as pl


def _mm_step(i, blk, bit, xa_ref, xb_ref, w1_ref, b1_ref, w2_ref, b2_ref,
             h_ref, tx_ref, out_ref):
    fa = jnp.maximum(
        jax.lax.dot_general(
            xa_ref[...], w1_ref[...], (((1,), (0,)), ((), ())),
            preferred_element_type=jnp.float32,
        ) + b1_ref[...], 0.0)
    fb = jnp.maximum(
        jax.lax.dot_general(
            xb_ref[...], w1_ref[...], (((1,), (0,)), ((), ())),
            preferred_element_type=jnp.float32,
        ) + b1_ref[...], 0.0)
    h_ref[pl.ds(i * blk, blk), :bit] = jax.lax.dot_general(
        fa, w2_ref[...], (((1,), (0,)), ((), ())),
        preferred_element_type=jnp.float32,
    ) + b2_ref[...]
    h_ref[pl.ds(i * blk, blk), bit:] = jax.lax.dot_general(
        fb, w2_ref[...], (((1,), (0,)), ((), ())),
        preferred_element_type=jnp.float32,
    ) + b2_ref[...]
    num = jnp.sum(fa * fb, axis=1, keepdims=True)
    na = jnp.maximum(jnp.sqrt(jnp.sum(fa * fa, axis=1, keepdims=True)), 1e-8)
    nb = jnp.maximum(jnp.sqrt(jnp.sum(fb * fb, axis=1, keepdims=True)), 1e-8)
    tx_ref[pl.ds(i * blk, blk), :] = num / (na * nb)


def _hash_loss_step(n2, bit, h_ref, tx_ref, out_ref):
    hp = h_ref[...]
    n = 2 * n2

    i32 = jax.lax.bitcast_convert_type(hp, jnp.int32)
    s32 = jnp.right_shift(i32, 31)
    key = (i32 ^ (s32 & jnp.int32(0x7FFFFFFF))) - s32

    int_min = jnp.int32(-(2**31))
    bkey = key ^ int_min
    k31 = jax.lax.shift_right_logical(bkey, 1)

    def fold(s):
        return s[:, :bit] + s[:, bit:]

    def tree(s):
        r = s.shape[0]
        while r > 8:
            r //= 2
            s = s[:r] + s[r:]
        return jnp.sum(s, axis=0, keepdims=True)

    def neg_count_lt(arr, p2):
        return fold(tree(jax.lax.shift_right_arithmetic(arr - p2, 31)))

    def both(v):
        return jnp.concatenate([v, v], axis=1)

    p = jnp.zeros((1, bit), jnp.int32)
    for bb in range(30, -1, -1):
        pp = p | jnp.int32(1 << bb)
        s = neg_count_lt(k31, both(pp))
        p = jnp.where(s >= -n2, pp, p)

    cnt_hi = n + neg_count_lt(k31, both(p + 1))
    eqm = jax.lax.shift_right_arithmetic((k31 ^ both(p)) - 1, 31)
    lsbm = -(bkey & 1)
    cnt_eq1 = -fold(tree(eqm & lsbm))
    lsb = jnp.where(cnt_hi + cnt_eq1 >= n2, jnp.int32(1), jnp.int32(0))
    thr = both((jnp.left_shift(p, 1) | lsb) ^ int_min)

    greater = key > thr
    g = jnp.where(lsb == 1, cnt_hi, cnt_hi + cnt_eq1)
    m = n2 - g
    eq = key == thr

    prow = jax.lax.broadcasted_iota(jnp.int32, (n2, 2 * bit), 0)
    lane = jax.lax.broadcasted_iota(jnp.int32, (n2, 2 * bit), 1)
    row = prow + jnp.where(lane >= bit, n2, 0)

    eqrow = jnp.where(eq, row, jnp.int32(1 << 30))

    def tie_min(s):
        r = s.shape[0]
        while r > 8:
            r //= 2
            s = jnp.minimum(s[:r], s[r:])
        s = jnp.min(s, axis=0, keepdims=True)
        return jnp.minimum(s[:, :bit], s[:, bit:])

    def tie_search(s):
        q = jnp.zeros((1, bit), jnp.int32)
        for bb in range(11, -1, -1):
            qp = q | jnp.int32(1 << bb)
            c = neg_count_lt(eqrow, both(qp))
            q = jnp.where(c > -m, qp, q)
        return q

    q = jax.lax.cond(jnp.all(m == 1), tie_min, tie_search, eqrow)

    plus = greater | (eq & (row <= both(q)))
    ba = jnp.where(plus[:, :bit], 1.0, -1.0)
    bb_ = jnp.where(plus[:, bit:], 1.0, -1.0)

    tb = jnp.sum(ba * bb_, axis=1, keepdims=True) / float(bit)

    diff = tb - tx_ref[...]
    out_ref[...] = jnp.sum(diff * diff, axis=0, keepdims=True) / float(n2)


def _fused_kernel(nblk, blk, n2, bit, xa_ref, xb_ref, w1_ref, b1_ref, w2_ref,
                  b2_ref, out_ref, h_ref, tx_ref):
    i = pl.program_id(0)

    @pl.when(i < nblk)
    def _():
        _mm_step(i, blk, bit, xa_ref, xb_ref, w1_ref, b1_ref, w2_ref, b2_ref,
                 h_ref, tx_ref, out_ref)

    @pl.when(i == nblk)
    def _():
        _hash_loss_step(n2, bit, h_ref, tx_ref, out_ref)


def kernel(x, W1, b1, W2, b2):
    n, d = x.shape
    hid = W1.shape[1]
    bit = W2.shape[1]
    n2 = n // 2
    blk = 512
    nblk = n2 // blk
    last = nblk - 1

    loss, _, _ = pl.pallas_call(
        functools.partial(_fused_kernel, nblk, blk, n2, bit),
        grid=(nblk + 1,),
        in_specs=[
            pl.BlockSpec((blk, d), lambda i: (jnp.minimum(i, last), 0)),
            pl.BlockSpec((blk, d), lambda i: (jnp.minimum(i, last) + nblk, 0)),
            pl.BlockSpec((d, hid), lambda i: (0, 0)),
            pl.BlockSpec((1, hid), lambda i: (0, 0)),
            pl.BlockSpec((hid, bit), lambda i: (0, 0)),
            pl.BlockSpec((1, bit), lambda i: (0, 0)),
        ],
        out_specs=[
            pl.BlockSpec((1, 1), lambda i: (0, 0)),
            pl.BlockSpec((n2, 2 * bit), lambda i: (0, 0)),
            pl.BlockSpec((n2, 1), lambda i: (0, 0)),
        ],
        out_shape=[
            jax.ShapeDtypeStruct((1, 1), jnp.float32),
            jax.ShapeDtypeStruct((n2, 2 * bit), jnp.float32),
            jax.ShapeDtypeStruct((n2, 1), jnp.float32),
        ],
    )(x, x, W1, b1.reshape(1, hid), W2, b2.reshape(1, bit))
    return loss[0, 0]
